# Initial kernel scaffold; baseline (speedup 1.0000x reference)
import jax
import jax.numpy as jnp
from jax.experimental import pallas as pl


def kernel(data, edge, embed, W_w, W_b, a, a2, b2):
    raise NotImplementedError("write your pallas kernel here")


# speedup vs baseline: 8.5818x; 8.5818x over previous
import functools
import math

import jax
import jax.numpy as jnp
from jax import lax
from jax.experimental import pallas as pl
from jax.experimental.pallas import tpu as pltpu
from jax.experimental.pallas import tpu_sc as plsc

N = 10000
E = 320000
IN_F = 128
HID = 64
HEADS = 4
ALPHA = 0.2
EPS = 1e-6
SCALE = 1.0 / math.sqrt(HID * HEADS)
F = HEADS * HID

NC = 2
NS = 16
CHUNK = 128
SEG = 20
NSEG = 8
CPT = SEG * NSEG
EPT = CHUNK * CPT
EPAD = EPT * NS
NP = 10240
NPT = NP // NS
WSEG = 2 * SEG * CHUNK

BN = 400


def _proj_body(x_ref, w_ref, b_ref, a12_ref, d_ref, s_ref):
    x = x_ref[...]
    d = jnp.dot(x, w_ref[...], preferred_element_type=jnp.float32) + b_ref[...]
    d_ref[0] = d[:, :IN_F]
    d_ref[1] = d[:, IN_F:]
    s = jnp.dot(d, a12_ref[...], preferred_element_type=jnp.float32)
    s_ref[0] = s[:, :4]
    s_ref[1] = s[:, 4:]


def _project(embed, wfull, bfull, a12):
    return pl.pallas_call(
        _proj_body,
        grid=(N // BN,),
        in_specs=[
            pl.BlockSpec((BN, IN_F), lambda i: (i, 0)),
            pl.BlockSpec((IN_F, F), lambda i: (0, 0)),
            pl.BlockSpec((1, F), lambda i: (0, 0)),
            pl.BlockSpec((F, 8), lambda i: (0, 0)),
        ],
        out_specs=[
            pl.BlockSpec((NC, BN, IN_F), lambda i: (0, i, 0)),
            pl.BlockSpec((NC, BN, 4), lambda i: (0, i, 0)),
        ],
        out_shape=[
            jax.ShapeDtypeStruct((NC, N, IN_F), jnp.float32),
            jax.ShapeDtypeStruct((NC, N, 4), jnp.float32),
        ],
    )(embed, wfull, bfull, a12)


def _wgt_body(s_hbm, e0_hbm, e1_hbm, w_out, s_v, e0s, e1s, wseg):
    c = lax.axis_index("c")
    s = lax.axis_index("s")
    iota16 = lax.iota(jnp.int32, 16)
    pltpu.sync_copy(s_hbm.at[c], s_v)

    def _seg(seg, carry):
        pltpu.sync_copy(e0_hbm.at[s, pl.ds(seg * SEG, SEG)], e0s)
        pltpu.sync_copy(e1_hbm.at[s, pl.ds(seg * SEG, SEG)], e1s)
        ebase = s * EPT + seg * (SEG * CHUNK)

        def _chunk(j, carry2):
            for g in range(CHUNK // 16):
                sl = pl.ds(g * 16, 16)
                e0g4 = e0s[j, 0, sl] * 4
                e1g4 = e1s[j, 0, sl] * 4
                l0 = (plsc.load_gather(s_v, [e0g4])
                      + plsc.load_gather(s_v, [e1g4 + 2]))
                l1 = (plsc.load_gather(s_v, [e0g4 + 1])
                      + plsc.load_gather(s_v, [e1g4 + 3]))
                l0 = jnp.where(l0 > 0, l0, ALPHA * l0) * SCALE
                l1 = jnp.where(l1 > 0, l1, ALPHA * l1) * SCALE
                pos = ebase + j * CHUNK + g * 16 + iota16
                valid = pos < E
                w0 = jnp.where(valid, jnp.exp(l0), 0.0)
                w1 = jnp.where(valid, jnp.exp(l1), 0.0)
                ids2 = (j * CHUNK + g * 16 + iota16) * 2
                plsc.store_scatter(wseg, [ids2], w0)
                plsc.store_scatter(wseg, [ids2 + 1], w1)
            return carry2
        lax.fori_loop(0, SEG, _chunk, 0)
        pltpu.sync_copy(wseg, w_out.at[c, pl.ds(2 * ebase, WSEG)])
        return carry
    lax.fori_loop(0, NSEG, _seg, 0)


@functools.cache
def _wgt_kernel():
    return pl.kernel(
        _wgt_body,
        out_type=jax.ShapeDtypeStruct((NC, 2 * EPAD), jnp.float32),
        mesh=plsc.VectorSubcoreMesh(core_axis_name="c", subcore_axis_name="s",
                                    num_cores=NC, num_subcores=NS),
        compiler_params=pltpu.CompilerParams(needs_layout_passes=False),
        scratch_types=[
            pltpu.VMEM((4 * N,), jnp.float32),
            pltpu.VMEM((SEG, 1, CHUNK), jnp.int32),
            pltpu.VMEM((SEG, 1, CHUNK), jnp.int32),
            pltpu.VMEM((WSEG,), jnp.float32),
        ],
    )


def _agg_body(d_hbm, w_hbm, e0_hbm, e1_hbm, hp_out, rs_out,
              e0s, e1s, rows, wseg, ibuf, zr1, hp_acc, rs_acc, sem):
    c = lax.axis_index("c")
    s = lax.axis_index("s")
    off = c * N
    row0 = s * NPT
    z16 = jnp.zeros((16,), jnp.float32)
    iota16 = lax.iota(jnp.int32, 16)

    def _zrows(i, carry):
        for k in range(IN_F // 16):
            rows[i, pl.ds(k * 16, 16)] = z16
        return carry
    lax.fori_loop(0, CHUNK, _zrows, 0)

    def _zr1(i, carry):
        zr1[pl.ds(i * 16, 16)] = z16
        return carry
    lax.fori_loop(0, 2 * NPT // 16, _zr1, 0)

    for q in range(NPT // CHUNK):
        pltpu.sync_copy(rows, hp_acc.at[pl.ds(row0 + q * CHUNK, CHUNK)])
    pltpu.sync_copy(zr1, rs_acc.at[pl.ds(2 * row0, 2 * NPT)])
    plsc.subcore_barrier()

    def _seg(seg, carry):
        pltpu.sync_copy(e0_hbm.at[s, pl.ds(seg * SEG, SEG)], e0s)
        pltpu.sync_copy(e1_hbm.at[s, pl.ds(seg * SEG, SEG)], e1s)
        ebase = s * EPT + seg * (SEG * CHUNK)
        pltpu.sync_copy(w_hbm.at[c, pl.ds(2 * ebase, WSEG)],
                        wseg.at[pl.ds(0, WSEG)])

        def _adj(j, carry2):
            for k in range(CHUNK // 16):
                sl = pl.ds(k * 16, 16)
                e1s[j, 0, sl] = e1s[j, 0, sl] + off
            return carry2
        lax.fori_loop(0, SEG, _adj, 0)

        def _chunk(j, carry2):
            pltpu.async_copy(d_hbm.at[e1s.at[j, 0]], rows, sem).wait()

            for g in range(CHUNK // 16):
                sl = pl.ds(g * 16, 16)
                ids2 = (g * 16 + iota16) * 2
                e0g2 = e0s[j, 0, sl] * 2
                plsc.store_scatter(ibuf, [ids2], e0g2)
                plsc.store_scatter(ibuf, [ids2 + 1], e0g2 + 1)

            def _mul(e, carry3):
                wv = wseg[pl.ds(2 * (j * CHUNK + e), 16)]
                w0s = wv[0]
                w1s = wv[1]
                for k in range(4):
                    sl = pl.ds(k * 16, 16)
                    rows[e, sl] = rows[e, sl] * w0s
                for k in range(4, 8):
                    sl = pl.ds(k * 16, 16)
                    rows[e, sl] = rows[e, sl] * w1s
                return carry3
            lax.fori_loop(0, CHUNK, _mul, 0)

            pltpu.sync_copy(rows, hp_acc.at[e0s.at[j, 0]], add=True)
            pltpu.sync_copy(wseg.at[pl.ds(2 * (j * CHUNK), 2 * CHUNK)],
                            rs_acc.at[ibuf], add=True)
            return carry2
        lax.fori_loop(0, SEG, _chunk, 0)
        return carry
    lax.fori_loop(0, NSEG, _seg, 0)

    plsc.subcore_barrier()

    for q in range(NPT // CHUNK):
        r0 = row0 + q * CHUNK
        pltpu.sync_copy(hp_acc.at[pl.ds(r0, CHUNK)], rows)
        pltpu.sync_copy(rows, hp_out.at[c, pl.ds(r0, CHUNK)])
    pltpu.sync_copy(rs_acc.at[pl.ds(2 * row0, 2 * NPT)], zr1)
    pltpu.sync_copy(zr1, rs_out.at[c, pl.ds(2 * row0, 2 * NPT)])


@functools.cache
def _agg_kernel():
    return pl.kernel(
        _agg_body,
        out_type=[
            jax.ShapeDtypeStruct((NC, NP, IN_F), jnp.float32),
            jax.ShapeDtypeStruct((NC, 2 * NP), jnp.float32),
        ],
        mesh=plsc.VectorSubcoreMesh(core_axis_name="c", subcore_axis_name="s",
                                    num_cores=NC, num_subcores=NS),
        compiler_params=pltpu.CompilerParams(needs_layout_passes=False),
        scratch_types=[
            pltpu.VMEM((SEG, 1, CHUNK), jnp.int32),
            pltpu.VMEM((SEG, 1, CHUNK), jnp.int32),
            pltpu.VMEM((CHUNK, IN_F), jnp.float32),
            pltpu.VMEM((WSEG + 16,), jnp.float32),
            pltpu.VMEM((2 * CHUNK,), jnp.int32),
            pltpu.VMEM((2 * NPT,), jnp.float32),
            pltpu.VMEM_SHARED((NP, IN_F), jnp.float32),
            pltpu.VMEM_SHARED((2 * NP,), jnp.float32),
            pltpu.SemaphoreType.DMA,
        ],
    )


def _ln_body(hp_ref, rs_ref, g_ref, b_ref, o_ref):
    hp0 = hp_ref[0]
    hp1 = hp_ref[1]
    rs = rs_ref[...]

    def _den(r):
        return jnp.where(r == 0.0, 1.0, r)

    h = jnp.concatenate([
        hp0[:, :HID] / _den(rs[0, :, 0:1]),
        hp0[:, HID:] / _den(rs[0, :, 1:2]),
        hp1[:, :HID] / _den(rs[1, :, 0:1]),
        hp1[:, HID:] / _den(rs[1, :, 1:2]),
    ], axis=1)
    mean = jnp.mean(h, axis=1, keepdims=True)
    xc = h - mean
    std = jnp.sqrt(jnp.sum(xc * xc, axis=1, keepdims=True) / (F - 1))
    y = g_ref[...] * xc / (std + EPS) + b_ref[...]
    o_ref[...] = jnp.where(y > 0, y, jnp.exp(jnp.minimum(y, 0.0)) - 1.0)


def _layernorm(hp, rs, gamma, beta):
    return pl.pallas_call(
        _ln_body,
        grid=(N // BN,),
        in_specs=[
            pl.BlockSpec((NC, BN, IN_F), lambda i: (0, i, 0)),
            pl.BlockSpec((NC, BN, 2), lambda i: (0, i, 0)),
            pl.BlockSpec((1, F), lambda i: (0, 0)),
            pl.BlockSpec((1, F), lambda i: (0, 0)),
        ],
        out_specs=pl.BlockSpec((BN, F), lambda i: (i, 0)),
        out_shape=jax.ShapeDtypeStruct((N, F), jnp.float32),
    )(hp, rs, gamma, beta)


def kernel(data, edge, embed, W_w, W_b, a, a2, b2):
    wfull = W_w.transpose(2, 0, 1).reshape(IN_F, F)
    bfull = W_b.reshape(1, F)
    a1 = a[:, 0, :HID]
    a2h = a[:, 0, HID:]
    cols = []
    for c in range(NC):
        for h in (2 * c, 2 * c + 1):
            cols.append(jnp.zeros((F,), jnp.float32).at[h * HID:(h + 1) * HID].set(a1[h]))
        for h in (2 * c, 2 * c + 1):
            cols.append(jnp.zeros((F,), jnp.float32).at[h * HID:(h + 1) * HID].set(a2h[h]))
    a12 = jnp.stack(cols, axis=1)

    e0 = jnp.zeros((EPAD,), jnp.int32).at[:E].set(edge[0])
    e1 = jnp.zeros((EPAD,), jnp.int32).at[:E].set(edge[1])
    e0r = e0.reshape(NS, CPT, 1, CHUNK)
    e1r = e1.reshape(NS, CPT, 1, CHUNK)

    d_out, s_out = _project(embed, wfull, bfull, a12)
    dcat = d_out.reshape(NC * N, IN_F)

    w_edge = _wgt_kernel()(s_out.reshape(NC, 4 * N), e0r, e1r)
    hp, rs = _agg_kernel()(dcat, w_edge, e0r, e1r)
    return _layernorm(hp, rs.reshape(NC, NP, 2),
                      a2.reshape(1, F), b2.reshape(1, F))


# speedup vs baseline: 11.1626x; 1.3007x over previous
import functools
import math

import jax
import jax.numpy as jnp
from jax import lax
from jax.experimental import pallas as pl
from jax.experimental.pallas import tpu as pltpu
from jax.experimental.pallas import tpu_sc as plsc

N = 10000
E = 320000
IN_F = 128
HID = 64
HEADS = 4
ALPHA = 0.2
EPS = 1e-6
SCALE = 1.0 / math.sqrt(HID * HEADS)
F = HEADS * HID

NC = 2
NS = 16
CHUNK = 128
SEG = 20
NSEG = 8
CPT = SEG * NSEG
EPT = CHUNK * CPT
EPAD = EPT * NS
NP = 10240
NPT = NP // NS
WSEG = 2 * SEG * CHUNK

BN = 400


def _proj_body(x_ref, w_ref, b_ref, a12_ref, d_ref, s_ref):
    x = x_ref[...]
    d = jnp.dot(x, w_ref[...], preferred_element_type=jnp.float32) + b_ref[...]
    d_ref[0] = d[:, :IN_F]
    d_ref[1] = d[:, IN_F:]
    s = jnp.dot(d, a12_ref[...], preferred_element_type=jnp.float32)
    s_ref[0] = s[:, :4]
    s_ref[1] = s[:, 4:]


def _project(embed, wfull, bfull, a12):
    return pl.pallas_call(
        _proj_body,
        grid=(N // BN,),
        in_specs=[
            pl.BlockSpec((BN, IN_F), lambda i: (i, 0)),
            pl.BlockSpec((IN_F, F), lambda i: (0, 0)),
            pl.BlockSpec((1, F), lambda i: (0, 0)),
            pl.BlockSpec((F, 8), lambda i: (0, 0)),
        ],
        out_specs=[
            pl.BlockSpec((NC, BN, IN_F), lambda i: (0, i, 0)),
            pl.BlockSpec((NC, BN, 4), lambda i: (0, i, 0)),
        ],
        out_shape=[
            jax.ShapeDtypeStruct((NC, N, IN_F), jnp.float32),
            jax.ShapeDtypeStruct((NC, N, 4), jnp.float32),
        ],
    )(embed, wfull, bfull, a12)


def _wgt_body(s_hbm, e0_hbm, e1_hbm, w_out, s_v, e0s, e1s, wseg):
    c = lax.axis_index("c")
    s = lax.axis_index("s")
    iota16 = lax.iota(jnp.int32, 16)
    pltpu.sync_copy(s_hbm.at[c], s_v)

    def _seg(seg, carry):
        pltpu.sync_copy(e0_hbm.at[s, pl.ds(seg * SEG, SEG)], e0s)
        pltpu.sync_copy(e1_hbm.at[s, pl.ds(seg * SEG, SEG)], e1s)
        ebase = s * EPT + seg * (SEG * CHUNK)

        def _chunk(j, carry2):
            for g in range(CHUNK // 16):
                sl = pl.ds(g * 16, 16)
                e0g4 = e0s[j, 0, sl] * 4
                e1g4 = e1s[j, 0, sl] * 4
                l0 = (plsc.load_gather(s_v, [e0g4])
                      + plsc.load_gather(s_v, [e1g4 + 2]))
                l1 = (plsc.load_gather(s_v, [e0g4 + 1])
                      + plsc.load_gather(s_v, [e1g4 + 3]))
                l0 = jnp.where(l0 > 0, l0, ALPHA * l0) * SCALE
                l1 = jnp.where(l1 > 0, l1, ALPHA * l1) * SCALE
                pos = ebase + j * CHUNK + g * 16 + iota16
                valid = pos < E
                w0 = jnp.where(valid, jnp.exp(l0), 0.0)
                w1 = jnp.where(valid, jnp.exp(l1), 0.0)
                ids2 = (j * CHUNK + g * 16 + iota16) * 2
                plsc.store_scatter(wseg, [ids2], w0)
                plsc.store_scatter(wseg, [ids2 + 1], w1)
            return carry2
        lax.fori_loop(0, SEG, _chunk, 0)
        pltpu.sync_copy(wseg, w_out.at[c, pl.ds(2 * ebase, WSEG)])
        return carry
    lax.fori_loop(0, NSEG, _seg, 0)


@functools.cache
def _wgt_kernel():
    return pl.kernel(
        _wgt_body,
        out_type=jax.ShapeDtypeStruct((NC, 2 * EPAD), jnp.float32),
        mesh=plsc.VectorSubcoreMesh(core_axis_name="c", subcore_axis_name="s",
                                    num_cores=NC, num_subcores=NS),
        compiler_params=pltpu.CompilerParams(needs_layout_passes=False),
        scratch_types=[
            pltpu.VMEM((4 * N,), jnp.float32),
            pltpu.VMEM((SEG, 1, CHUNK), jnp.int32),
            pltpu.VMEM((SEG, 1, CHUNK), jnp.int32),
            pltpu.VMEM((WSEG,), jnp.float32),
        ],
    )


def _agg_body(d_hbm, w_hbm, e0_hbm, e1_hbm, hp_out, rs_out,
              e0s, e1s, rows, rows2, wseg, ibuf, zr1, hp_acc, rs_acc, sem, sem2):
    c = lax.axis_index("c")
    s = lax.axis_index("s")
    off = c * N
    row0 = s * NPT
    z16 = jnp.zeros((16,), jnp.float32)
    iota16 = lax.iota(jnp.int32, 16)

    def _zrows(i, carry):
        for k in range(IN_F // 16):
            rows[i, pl.ds(k * 16, 16)] = z16
        return carry
    lax.fori_loop(0, CHUNK, _zrows, 0)

    def _zr1(i, carry):
        zr1[pl.ds(i * 16, 16)] = z16
        return carry
    lax.fori_loop(0, 2 * NPT // 16, _zr1, 0)

    for q in range(NPT // CHUNK):
        pltpu.sync_copy(rows, hp_acc.at[pl.ds(row0 + q * CHUNK, CHUNK)])
    pltpu.sync_copy(zr1, rs_acc.at[pl.ds(2 * row0, 2 * NPT)])
    plsc.subcore_barrier()

    def _seg(seg, carry):
        pltpu.sync_copy(e0_hbm.at[s, pl.ds(seg * SEG, SEG)], e0s)
        pltpu.sync_copy(e1_hbm.at[s, pl.ds(seg * SEG, SEG)], e1s)
        ebase = s * EPT + seg * (SEG * CHUNK)
        pltpu.sync_copy(w_hbm.at[c, pl.ds(2 * ebase, WSEG)],
                        wseg.at[pl.ds(0, WSEG)])

        def _adj(j, carry2):
            for k in range(CHUNK // 16):
                sl = pl.ds(k * 16, 16)
                e1s[j, 0, sl] = e1s[j, 0, sl] + off
            return carry2
        lax.fori_loop(0, SEG, _adj, 0)

        def _compute_scatter(j, buf):
            for g in range(CHUNK // 16):
                sl = pl.ds(g * 16, 16)
                ids2 = (g * 16 + iota16) * 2
                e0g2 = e0s[j, 0, sl] * 2
                plsc.store_scatter(ibuf, [ids2], e0g2)
                plsc.store_scatter(ibuf, [ids2 + 1], e0g2 + 1)

            def _mul(e, carry3):
                wv = wseg[pl.ds(2 * (j * CHUNK + e), 16)]
                w0s = wv[0]
                w1s = wv[1]
                for k in range(4):
                    sl = pl.ds(k * 16, 16)
                    buf[e, sl] = buf[e, sl] * w0s
                for k in range(4, 8):
                    sl = pl.ds(k * 16, 16)
                    buf[e, sl] = buf[e, sl] * w1s
                return carry3
            lax.fori_loop(0, CHUNK, _mul, 0, unroll=4)

            pltpu.sync_copy(buf, hp_acc.at[e0s.at[j, 0]], add=True)
            pltpu.sync_copy(wseg.at[pl.ds(2 * (j * CHUNK), 2 * CHUNK)],
                            rs_acc.at[ibuf], add=True)

        def _gather(j, buf, gsem):
            pltpu.async_copy(d_hbm.at[e1s.at[j, 0]], buf, gsem)

        def _gwait(buf, gsem):
            pltpu.make_async_copy(d_hbm.at[e1s.at[0, 0]], buf, gsem).wait()

        _gather(0, rows, sem)

        def _pair(p, carry2):
            j0 = 2 * p
            j1 = j0 + 1
            _gwait(rows, sem)
            _gather(j1, rows2, sem2)
            _compute_scatter(j0, rows)
            _gwait(rows2, sem2)

            @pl.when(j1 + 1 < SEG)
            def _():
                _gather(j1 + 1, rows, sem)
            _compute_scatter(j1, rows2)
            return carry2
        lax.fori_loop(0, SEG // 2, _pair, 0)
        return carry
    lax.fori_loop(0, NSEG, _seg, 0)

    plsc.subcore_barrier()

    for q in range(NPT // CHUNK):
        r0 = row0 + q * CHUNK
        pltpu.sync_copy(hp_acc.at[pl.ds(r0, CHUNK)], rows)
        pltpu.sync_copy(rows, hp_out.at[c, pl.ds(r0, CHUNK)])
    pltpu.sync_copy(rs_acc.at[pl.ds(2 * row0, 2 * NPT)], zr1)
    pltpu.sync_copy(zr1, rs_out.at[c, pl.ds(2 * row0, 2 * NPT)])


@functools.cache
def _agg_kernel():
    return pl.kernel(
        _agg_body,
        out_type=[
            jax.ShapeDtypeStruct((NC, NP, IN_F), jnp.float32),
            jax.ShapeDtypeStruct((NC, 2 * NP), jnp.float32),
        ],
        mesh=plsc.VectorSubcoreMesh(core_axis_name="c", subcore_axis_name="s",
                                    num_cores=NC, num_subcores=NS),
        compiler_params=pltpu.CompilerParams(needs_layout_passes=False),
        scratch_types=[
            pltpu.VMEM((SEG, 1, CHUNK), jnp.int32),
            pltpu.VMEM((SEG, 1, CHUNK), jnp.int32),
            pltpu.VMEM((CHUNK, IN_F), jnp.float32),
            pltpu.VMEM((CHUNK, IN_F), jnp.float32),
            pltpu.VMEM((WSEG + 16,), jnp.float32),
            pltpu.VMEM((2 * CHUNK,), jnp.int32),
            pltpu.VMEM((2 * NPT,), jnp.float32),
            pltpu.VMEM_SHARED((NP, IN_F), jnp.float32),
            pltpu.VMEM_SHARED((2 * NP,), jnp.float32),
            pltpu.SemaphoreType.DMA,
            pltpu.SemaphoreType.DMA,
        ],
    )


def _ln_body(hp_ref, rs_ref, g_ref, b_ref, o_ref):
    hp0 = hp_ref[0]
    hp1 = hp_ref[1]
    rs = rs_ref[...]

    def _den(r):
        return jnp.where(r == 0.0, 1.0, r)

    h = jnp.concatenate([
        hp0[:, :HID] / _den(rs[0, :, 0:1]),
        hp0[:, HID:] / _den(rs[0, :, 1:2]),
        hp1[:, :HID] / _den(rs[1, :, 0:1]),
        hp1[:, HID:] / _den(rs[1, :, 1:2]),
    ], axis=1)
    mean = jnp.mean(h, axis=1, keepdims=True)
    xc = h - mean
    std = jnp.sqrt(jnp.sum(xc * xc, axis=1, keepdims=True) / (F - 1))
    y = g_ref[...] * xc / (std + EPS) + b_ref[...]
    o_ref[...] = jnp.where(y > 0, y, jnp.exp(jnp.minimum(y, 0.0)) - 1.0)


def _layernorm(hp, rs, gamma, beta):
    return pl.pallas_call(
        _ln_body,
        grid=(N // BN,),
        in_specs=[
            pl.BlockSpec((NC, BN, IN_F), lambda i: (0, i, 0)),
            pl.BlockSpec((NC, BN, 2), lambda i: (0, i, 0)),
            pl.BlockSpec((1, F), lambda i: (0, 0)),
            pl.BlockSpec((1, F), lambda i: (0, 0)),
        ],
        out_specs=pl.BlockSpec((BN, F), lambda i: (i, 0)),
        out_shape=jax.ShapeDtypeStruct((N, F), jnp.float32),
    )(hp, rs, gamma, beta)


def kernel(data, edge, embed, W_w, W_b, a, a2, b2):
    wfull = W_w.transpose(2, 0, 1).reshape(IN_F, F)
    bfull = W_b.reshape(1, F)
    a1 = a[:, 0, :HID]
    a2h = a[:, 0, HID:]
    cols = []
    for c in range(NC):
        for h in (2 * c, 2 * c + 1):
            cols.append(jnp.zeros((F,), jnp.float32).at[h * HID:(h + 1) * HID].set(a1[h]))
        for h in (2 * c, 2 * c + 1):
            cols.append(jnp.zeros((F,), jnp.float32).at[h * HID:(h + 1) * HID].set(a2h[h]))
    a12 = jnp.stack(cols, axis=1)

    e0 = jnp.zeros((EPAD,), jnp.int32).at[:E].set(edge[0])
    e1 = jnp.zeros((EPAD,), jnp.int32).at[:E].set(edge[1])
    e0r = e0.reshape(NS, CPT, 1, CHUNK)
    e1r = e1.reshape(NS, CPT, 1, CHUNK)

    d_out, s_out = _project(embed, wfull, bfull, a12)
    dcat = d_out.reshape(NC * N, IN_F)

    w_edge = _wgt_kernel()(s_out.reshape(NC, 4 * N), e0r, e1r)
    hp, rs = _agg_kernel()(dcat, w_edge, e0r, e1r)
    return _layernorm(hp, rs.reshape(NC, NP, 2),
                      a2.reshape(1, F), b2.reshape(1, F))


# speedup vs baseline: 11.2974x; 1.0121x over previous
import functools
import math

import jax
import jax.numpy as jnp
from jax import lax
from jax.experimental import pallas as pl
from jax.experimental.pallas import tpu as pltpu
from jax.experimental.pallas import tpu_sc as plsc

N = 10000
E = 320000
IN_F = 128
HID = 64
HEADS = 4
ALPHA = 0.2
EPS = 1e-6
SCALE = 1.0 / math.sqrt(HID * HEADS)
F = HEADS * HID

NC = 2
NS = 16
CHUNK = 128
SEG = 20
NSEG = 8
CPT = SEG * NSEG
EPT = CHUNK * CPT
EPAD = EPT * NS
NP = 10240
NPT = NP // NS
WSEG = 2 * SEG * CHUNK

BN = 400


def _proj_body(x_ref, w_ref, b_ref, a12_ref, d_ref, s_ref):
    x = x_ref[...]
    d = jnp.dot(x, w_ref[...], preferred_element_type=jnp.float32) + b_ref[...]
    d_ref[0] = d[:, :IN_F]
    d_ref[1] = d[:, IN_F:]
    s = jnp.dot(d, a12_ref[...], preferred_element_type=jnp.float32)
    s_ref[0] = s[:, :4]
    s_ref[1] = s[:, 4:]


def _project(embed, wfull, bfull, a12):
    return pl.pallas_call(
        _proj_body,
        grid=(N // BN,),
        in_specs=[
            pl.BlockSpec((BN, IN_F), lambda i: (i, 0)),
            pl.BlockSpec((IN_F, F), lambda i: (0, 0)),
            pl.BlockSpec((1, F), lambda i: (0, 0)),
            pl.BlockSpec((F, 8), lambda i: (0, 0)),
        ],
        out_specs=[
            pl.BlockSpec((NC, BN, IN_F), lambda i: (0, i, 0)),
            pl.BlockSpec((NC, BN, 4), lambda i: (0, i, 0)),
        ],
        out_shape=[
            jax.ShapeDtypeStruct((NC, N, IN_F), jnp.float32),
            jax.ShapeDtypeStruct((NC, N, 4), jnp.float32),
        ],
    )(embed, wfull, bfull, a12)


def _wgt_body(s_hbm, e0_hbm, e1_hbm, w_out, s_v, e0s, e1s, wseg):
    c = lax.axis_index("c")
    s = lax.axis_index("s")
    iota16 = lax.iota(jnp.int32, 16)
    pltpu.sync_copy(s_hbm.at[c], s_v)

    def _seg(seg, carry):
        pltpu.sync_copy(e0_hbm.at[s, pl.ds(seg * SEG, SEG)], e0s)
        pltpu.sync_copy(e1_hbm.at[s, pl.ds(seg * SEG, SEG)], e1s)
        ebase = s * EPT + seg * (SEG * CHUNK)

        def _chunk(j, carry2):
            for g in range(CHUNK // 16):
                sl = pl.ds(g * 16, 16)
                e0g4 = e0s[j, 0, sl] * 4
                e1g4 = e1s[j, 0, sl] * 4
                l0 = (plsc.load_gather(s_v, [e0g4])
                      + plsc.load_gather(s_v, [e1g4 + 2]))
                l1 = (plsc.load_gather(s_v, [e0g4 + 1])
                      + plsc.load_gather(s_v, [e1g4 + 3]))
                l0 = jnp.where(l0 > 0, l0, ALPHA * l0) * SCALE
                l1 = jnp.where(l1 > 0, l1, ALPHA * l1) * SCALE
                pos = ebase + j * CHUNK + g * 16 + iota16
                valid = pos < E
                w0 = jnp.where(valid, jnp.exp(l0), 0.0)
                w1 = jnp.where(valid, jnp.exp(l1), 0.0)
                ids2 = (j * CHUNK + g * 16 + iota16) * 2
                plsc.store_scatter(wseg, [ids2], w0)
                plsc.store_scatter(wseg, [ids2 + 1], w1)
            return carry2
        lax.fori_loop(0, SEG, _chunk, 0)
        pltpu.sync_copy(wseg, w_out.at[c, pl.ds(2 * ebase, WSEG)])
        return carry
    lax.fori_loop(0, NSEG, _seg, 0)


@functools.cache
def _wgt_kernel():
    return pl.kernel(
        _wgt_body,
        out_type=jax.ShapeDtypeStruct((NC, 2 * EPAD), jnp.float32),
        mesh=plsc.VectorSubcoreMesh(core_axis_name="c", subcore_axis_name="s",
                                    num_cores=NC, num_subcores=NS),
        compiler_params=pltpu.CompilerParams(needs_layout_passes=False),
        scratch_types=[
            pltpu.VMEM((4 * N,), jnp.float32),
            pltpu.VMEM((SEG, 1, CHUNK), jnp.int32),
            pltpu.VMEM((SEG, 1, CHUNK), jnp.int32),
            pltpu.VMEM((WSEG,), jnp.float32),
        ],
    )


def _agg_body(d_hbm, w_hbm, e0_hbm, e1_hbm, hp_out, rs_out,
              e0s, e1s, rows, rows2, wseg, ibuf, zr1, hp_acc, rs_acc,
              sem, sem2, ssem, ssem2):
    c = lax.axis_index("c")
    s = lax.axis_index("s")
    off = c * N
    row0 = s * NPT
    z16 = jnp.zeros((16,), jnp.float32)
    iota16 = lax.iota(jnp.int32, 16)

    def _zrows(i, carry):
        for k in range(IN_F // 16):
            rows[i, pl.ds(k * 16, 16)] = z16
        return carry
    lax.fori_loop(0, CHUNK, _zrows, 0)

    def _zr1(i, carry):
        zr1[pl.ds(i * 16, 16)] = z16
        return carry
    lax.fori_loop(0, 2 * NPT // 16, _zr1, 0)

    for q in range(NPT // CHUNK):
        pltpu.sync_copy(rows, hp_acc.at[pl.ds(row0 + q * CHUNK, CHUNK)])
    pltpu.sync_copy(zr1, rs_acc.at[pl.ds(2 * row0, 2 * NPT)])
    plsc.subcore_barrier()

    def _seg(seg, carry):
        pltpu.sync_copy(e0_hbm.at[s, pl.ds(seg * SEG, SEG)], e0s)
        pltpu.sync_copy(e1_hbm.at[s, pl.ds(seg * SEG, SEG)], e1s)
        ebase = s * EPT + seg * (SEG * CHUNK)
        pltpu.sync_copy(w_hbm.at[c, pl.ds(2 * ebase, WSEG)],
                        wseg.at[pl.ds(0, WSEG)])

        def _adj(j, carry2):
            for k in range(CHUNK // 16):
                sl = pl.ds(k * 16, 16)
                e1s[j, 0, sl] = e1s[j, 0, sl] + off
            return carry2
        lax.fori_loop(0, SEG, _adj, 0)

        def _compute_scatter(j, buf, ssem):
            for g in range(CHUNK // 16):
                sl = pl.ds(g * 16, 16)
                ids2 = (g * 16 + iota16) * 2
                e0g2 = e0s[j, 0, sl] * 2
                plsc.store_scatter(ibuf, [ids2], e0g2)
                plsc.store_scatter(ibuf, [ids2 + 1], e0g2 + 1)

            def _mul(e, carry3):
                wv = wseg[pl.ds(2 * (j * CHUNK + e), 16)]
                w0s = wv[0]
                w1s = wv[1]
                for k in range(4):
                    sl = pl.ds(k * 16, 16)
                    buf[e, sl] = buf[e, sl] * w0s
                for k in range(4, 8):
                    sl = pl.ds(k * 16, 16)
                    buf[e, sl] = buf[e, sl] * w1s
                return carry3
            lax.fori_loop(0, CHUNK, _mul, 0, unroll=4)

            pltpu.async_copy(buf, hp_acc.at[e0s.at[j, 0]], ssem, add=True)
            pltpu.sync_copy(wseg.at[pl.ds(2 * (j * CHUNK), 2 * CHUNK)],
                            rs_acc.at[ibuf], add=True)

        def _gather(j, buf, gsem):
            pltpu.async_copy(d_hbm.at[e1s.at[j, 0]], buf, gsem)

        def _gwait(buf, gsem):
            pltpu.make_async_copy(d_hbm.at[e1s.at[0, 0]], buf, gsem).wait()

        def _swait(buf, xsem):
            pltpu.make_async_copy(buf, hp_acc.at[e0s.at[0, 0]], xsem).wait()

        _gather(0, rows, sem)
        _gather(1, rows2, sem2)

        def _pair(p, carry2):
            j0 = 2 * p
            j1 = j0 + 1
            _gwait(rows, sem)
            _compute_scatter(j0, rows, ssem)
            _gwait(rows2, sem2)
            _swait(rows, ssem)

            @pl.when(j0 + 2 < SEG)
            def _():
                _gather(j0 + 2, rows, sem)
            _compute_scatter(j1, rows2, ssem2)
            _swait(rows2, ssem2)

            @pl.when(j1 + 2 < SEG)
            def _():
                _gather(j1 + 2, rows2, sem2)
            return carry2
        lax.fori_loop(0, SEG // 2, _pair, 0)
        return carry
    lax.fori_loop(0, NSEG, _seg, 0)

    plsc.subcore_barrier()

    for q in range(NPT // CHUNK):
        r0 = row0 + q * CHUNK
        pltpu.sync_copy(hp_acc.at[pl.ds(r0, CHUNK)], rows)
        pltpu.sync_copy(rows, hp_out.at[c, pl.ds(r0, CHUNK)])
    pltpu.sync_copy(rs_acc.at[pl.ds(2 * row0, 2 * NPT)], zr1)
    pltpu.sync_copy(zr1, rs_out.at[c, pl.ds(2 * row0, 2 * NPT)])


@functools.cache
def _agg_kernel():
    return pl.kernel(
        _agg_body,
        out_type=[
            jax.ShapeDtypeStruct((NC, NP, IN_F), jnp.float32),
            jax.ShapeDtypeStruct((NC, 2 * NP), jnp.float32),
        ],
        mesh=plsc.VectorSubcoreMesh(core_axis_name="c", subcore_axis_name="s",
                                    num_cores=NC, num_subcores=NS),
        compiler_params=pltpu.CompilerParams(needs_layout_passes=False),
        scratch_types=[
            pltpu.VMEM((SEG, 1, CHUNK), jnp.int32),
            pltpu.VMEM((SEG, 1, CHUNK), jnp.int32),
            pltpu.VMEM((CHUNK, IN_F), jnp.float32),
            pltpu.VMEM((CHUNK, IN_F), jnp.float32),
            pltpu.VMEM((WSEG + 16,), jnp.float32),
            pltpu.VMEM((2 * CHUNK,), jnp.int32),
            pltpu.VMEM((2 * NPT,), jnp.float32),
            pltpu.VMEM_SHARED((NP, IN_F), jnp.float32),
            pltpu.VMEM_SHARED((2 * NP,), jnp.float32),
            pltpu.SemaphoreType.DMA,
            pltpu.SemaphoreType.DMA,
            pltpu.SemaphoreType.DMA,
            pltpu.SemaphoreType.DMA,
        ],
    )


def _ln_body(hp_ref, rs_ref, g_ref, b_ref, o_ref):
    hp0 = hp_ref[0]
    hp1 = hp_ref[1]
    rs = rs_ref[...]

    def _den(r):
        return jnp.where(r == 0.0, 1.0, r)

    h = jnp.concatenate([
        hp0[:, :HID] / _den(rs[0, :, 0:1]),
        hp0[:, HID:] / _den(rs[0, :, 1:2]),
        hp1[:, :HID] / _den(rs[1, :, 0:1]),
        hp1[:, HID:] / _den(rs[1, :, 1:2]),
    ], axis=1)
    mean = jnp.mean(h, axis=1, keepdims=True)
    xc = h - mean
    std = jnp.sqrt(jnp.sum(xc * xc, axis=1, keepdims=True) / (F - 1))
    y = g_ref[...] * xc / (std + EPS) + b_ref[...]
    o_ref[...] = jnp.where(y > 0, y, jnp.exp(jnp.minimum(y, 0.0)) - 1.0)


def _layernorm(hp, rs, gamma, beta):
    return pl.pallas_call(
        _ln_body,
        grid=(N // BN,),
        in_specs=[
            pl.BlockSpec((NC, BN, IN_F), lambda i: (0, i, 0)),
            pl.BlockSpec((NC, BN, 2), lambda i: (0, i, 0)),
            pl.BlockSpec((1, F), lambda i: (0, 0)),
            pl.BlockSpec((1, F), lambda i: (0, 0)),
        ],
        out_specs=pl.BlockSpec((BN, F), lambda i: (i, 0)),
        out_shape=jax.ShapeDtypeStruct((N, F), jnp.float32),
    )(hp, rs, gamma, beta)


def kernel(data, edge, embed, W_w, W_b, a, a2, b2):
    wfull = W_w.transpose(2, 0, 1).reshape(IN_F, F)
    bfull = W_b.reshape(1, F)
    a1 = a[:, 0, :HID]
    a2h = a[:, 0, HID:]
    cols = []
    for c in range(NC):
        for h in (2 * c, 2 * c + 1):
            cols.append(jnp.zeros((F,), jnp.float32).at[h * HID:(h + 1) * HID].set(a1[h]))
        for h in (2 * c, 2 * c + 1):
            cols.append(jnp.zeros((F,), jnp.float32).at[h * HID:(h + 1) * HID].set(a2h[h]))
    a12 = jnp.stack(cols, axis=1)

    e0 = jnp.zeros((EPAD,), jnp.int32).at[:E].set(edge[0])
    e1 = jnp.zeros((EPAD,), jnp.int32).at[:E].set(edge[1])
    e0r = e0.reshape(NS, CPT, 1, CHUNK)
    e1r = e1.reshape(NS, CPT, 1, CHUNK)

    d_out, s_out = _project(embed, wfull, bfull, a12)
    dcat = d_out.reshape(NC * N, IN_F)

    w_edge = _wgt_kernel()(s_out.reshape(NC, 4 * N), e0r, e1r)
    hp, rs = _agg_kernel()(dcat, w_edge, e0r, e1r)
    return _layernorm(hp, rs.reshape(NC, NP, 2),
                      a2.reshape(1, F), b2.reshape(1, F))


# speedup vs baseline: 11.2999x; 1.0002x over previous
import functools
import math

import jax
import jax.numpy as jnp
from jax import lax
from jax.experimental import pallas as pl
from jax.experimental.pallas import tpu as pltpu
from jax.experimental.pallas import tpu_sc as plsc

N = 10000
E = 320000
IN_F = 128
HID = 64
HEADS = 4
ALPHA = 0.2
EPS = 1e-6
SCALE = 1.0 / math.sqrt(HID * HEADS)
F = HEADS * HID

NC = 2
NS = 16
CHUNK = 128
SEG = 20
NSEG = 8
CPT = SEG * NSEG
EPT = CHUNK * CPT
EPAD = EPT * NS
NP = 10240
NPT = NP // NS
WSEG = 2 * SEG * CHUNK

BN = 400


def _proj_body(x_ref, w_ref, b_ref, a12_ref, d_ref, s_ref):
    x = x_ref[...]
    d = jnp.dot(x, w_ref[...], preferred_element_type=jnp.float32) + b_ref[...]
    d_ref[0] = d[:, :IN_F]
    d_ref[1] = d[:, IN_F:]
    s = jnp.dot(d, a12_ref[...], preferred_element_type=jnp.float32)
    s_ref[0] = s[:, :4]
    s_ref[1] = s[:, 4:]


def _project(embed, wfull, bfull, a12):
    return pl.pallas_call(
        _proj_body,
        grid=(N // BN,),
        in_specs=[
            pl.BlockSpec((BN, IN_F), lambda i: (i, 0)),
            pl.BlockSpec((IN_F, F), lambda i: (0, 0)),
            pl.BlockSpec((1, F), lambda i: (0, 0)),
            pl.BlockSpec((F, 8), lambda i: (0, 0)),
        ],
        out_specs=[
            pl.BlockSpec((NC, BN, IN_F), lambda i: (0, i, 0)),
            pl.BlockSpec((NC, BN, 4), lambda i: (0, i, 0)),
        ],
        out_shape=[
            jax.ShapeDtypeStruct((NC, N, IN_F), jnp.float32),
            jax.ShapeDtypeStruct((NC, N, 4), jnp.float32),
        ],
    )(embed, wfull, bfull, a12)


def _wgt_body(s_hbm, e0_hbm, e1_hbm, w_out, s_v, e0s, e1s, wseg):
    c = lax.axis_index("c")
    s = lax.axis_index("s")
    iota16 = lax.iota(jnp.int32, 16)
    pltpu.sync_copy(s_hbm.at[c], s_v)

    def _seg(seg, carry):
        pltpu.sync_copy(e0_hbm.at[s, pl.ds(seg * SEG, SEG)], e0s)
        pltpu.sync_copy(e1_hbm.at[s, pl.ds(seg * SEG, SEG)], e1s)
        ebase = s * EPT + seg * (SEG * CHUNK)

        def _chunk(j, carry2):
            for g in range(CHUNK // 16):
                sl = pl.ds(g * 16, 16)
                e0g4 = e0s[j, 0, sl] * 4
                e1g4 = e1s[j, 0, sl] * 4
                l0 = (plsc.load_gather(s_v, [e0g4])
                      + plsc.load_gather(s_v, [e1g4 + 2]))
                l1 = (plsc.load_gather(s_v, [e0g4 + 1])
                      + plsc.load_gather(s_v, [e1g4 + 3]))
                l0 = jnp.where(l0 > 0, l0, ALPHA * l0) * SCALE
                l1 = jnp.where(l1 > 0, l1, ALPHA * l1) * SCALE
                pos = ebase + j * CHUNK + g * 16 + iota16
                valid = pos < E
                w0 = jnp.where(valid, jnp.exp(l0), 0.0)
                w1 = jnp.where(valid, jnp.exp(l1), 0.0)
                ids2 = (j * CHUNK + g * 16 + iota16) * 2
                plsc.store_scatter(wseg, [ids2], w0)
                plsc.store_scatter(wseg, [ids2 + 1], w1)
            return carry2
        lax.fori_loop(0, SEG, _chunk, 0)
        pltpu.sync_copy(wseg, w_out.at[c, pl.ds(2 * ebase, WSEG)])
        return carry
    lax.fori_loop(0, NSEG, _seg, 0)


@functools.cache
def _wgt_kernel():
    return pl.kernel(
        _wgt_body,
        out_type=jax.ShapeDtypeStruct((NC, 2 * EPAD), jnp.float32),
        mesh=plsc.VectorSubcoreMesh(core_axis_name="c", subcore_axis_name="s",
                                    num_cores=NC, num_subcores=NS),
        compiler_params=pltpu.CompilerParams(needs_layout_passes=False),
        scratch_types=[
            pltpu.VMEM((4 * N,), jnp.float32),
            pltpu.VMEM((SEG, 1, CHUNK), jnp.int32),
            pltpu.VMEM((SEG, 1, CHUNK), jnp.int32),
            pltpu.VMEM((WSEG,), jnp.float32),
        ],
    )


def _agg_body(d_hbm, w_hbm, e0_hbm, e1_hbm, hp_out, rs_out,
              e0s, e1s, rows, rows2, wseg, ibuf, zr1, hp_acc, rs_acc,
              sem, sem2, ssem, ssem2):
    c = lax.axis_index("c")
    s = lax.axis_index("s")
    off = c * N
    row0 = s * NPT
    z16 = jnp.zeros((16,), jnp.float32)
    iota16 = lax.iota(jnp.int32, 16)

    def _zrows(i, carry):
        for k in range(IN_F // 16):
            rows[i, pl.ds(k * 16, 16)] = z16
        return carry
    lax.fori_loop(0, CHUNK, _zrows, 0)

    def _zr1(i, carry):
        zr1[pl.ds(i * 16, 16)] = z16
        return carry
    lax.fori_loop(0, 2 * NPT // 16, _zr1, 0)

    for q in range(NPT // CHUNK):
        pltpu.sync_copy(rows, hp_acc.at[pl.ds(row0 + q * CHUNK, CHUNK)])
    pltpu.sync_copy(zr1, rs_acc.at[pl.ds(2 * row0, 2 * NPT)])
    plsc.subcore_barrier()

    def _seg(seg, carry):
        pltpu.sync_copy(e0_hbm.at[s, pl.ds(seg * SEG, SEG)], e0s)
        pltpu.sync_copy(e1_hbm.at[s, pl.ds(seg * SEG, SEG)], e1s)
        ebase = s * EPT + seg * (SEG * CHUNK)
        pltpu.sync_copy(w_hbm.at[c, pl.ds(2 * ebase, WSEG)],
                        wseg.at[pl.ds(0, WSEG)])

        def _adj(j, carry2):
            for k in range(CHUNK // 16):
                sl = pl.ds(k * 16, 16)
                e1s[j, 0, sl] = e1s[j, 0, sl] + off
            return carry2
        lax.fori_loop(0, SEG, _adj, 0)

        def _compute_scatter(j, buf, ssem):
            for g in range(CHUNK // 16):
                sl = pl.ds(g * 16, 16)
                ids2 = (g * 16 + iota16) * 2
                e0g2 = e0s[j, 0, sl] * 2
                plsc.store_scatter(ibuf, [ids2], e0g2)
                plsc.store_scatter(ibuf, [ids2 + 1], e0g2 + 1)

            def _mul(e, carry3):
                wv = wseg[pl.ds(2 * (j * CHUNK + e), 16)]
                w0s = wv[0]
                w1s = wv[1]
                for k in range(4):
                    sl = pl.ds(k * 16, 16)
                    buf[e, sl] = buf[e, sl] * w0s
                for k in range(4, 8):
                    sl = pl.ds(k * 16, 16)
                    buf[e, sl] = buf[e, sl] * w1s
                return carry3
            lax.fori_loop(0, CHUNK, _mul, 0, unroll=8)

            pltpu.async_copy(buf, hp_acc.at[e0s.at[j, 0]], ssem, add=True)
            pltpu.sync_copy(wseg.at[pl.ds(2 * (j * CHUNK), 2 * CHUNK)],
                            rs_acc.at[ibuf], add=True)

        def _gather(j, buf, gsem):
            pltpu.async_copy(d_hbm.at[e1s.at[j, 0]], buf, gsem)

        def _gwait(buf, gsem):
            pltpu.make_async_copy(d_hbm.at[e1s.at[0, 0]], buf, gsem).wait()

        def _swait(buf, xsem):
            pltpu.make_async_copy(buf, hp_acc.at[e0s.at[0, 0]], xsem).wait()

        _gather(0, rows, sem)
        _gather(1, rows2, sem2)

        def _pair(p, carry2):
            j0 = 2 * p
            j1 = j0 + 1
            _gwait(rows, sem)
            _compute_scatter(j0, rows, ssem)
            _gwait(rows2, sem2)
            _swait(rows, ssem)

            @pl.when(j0 + 2 < SEG)
            def _():
                _gather(j0 + 2, rows, sem)
            _compute_scatter(j1, rows2, ssem2)
            _swait(rows2, ssem2)

            @pl.when(j1 + 2 < SEG)
            def _():
                _gather(j1 + 2, rows2, sem2)
            return carry2
        lax.fori_loop(0, SEG // 2, _pair, 0)
        return carry
    lax.fori_loop(0, NSEG, _seg, 0)

    plsc.subcore_barrier()

    for q in range(NPT // CHUNK):
        r0 = row0 + q * CHUNK
        pltpu.sync_copy(hp_acc.at[pl.ds(r0, CHUNK)], rows)
        pltpu.sync_copy(rows, hp_out.at[c, pl.ds(r0, CHUNK)])
    pltpu.sync_copy(rs_acc.at[pl.ds(2 * row0, 2 * NPT)], zr1)
    pltpu.sync_copy(zr1, rs_out.at[c, pl.ds(2 * row0, 2 * NPT)])


@functools.cache
def _agg_kernel():
    return pl.kernel(
        _agg_body,
        out_type=[
            jax.ShapeDtypeStruct((NC, NP, IN_F), jnp.float32),
            jax.ShapeDtypeStruct((NC, 2 * NP), jnp.float32),
        ],
        mesh=plsc.VectorSubcoreMesh(core_axis_name="c", subcore_axis_name="s",
                                    num_cores=NC, num_subcores=NS),
        compiler_params=pltpu.CompilerParams(needs_layout_passes=False),
        scratch_types=[
            pltpu.VMEM((SEG, 1, CHUNK), jnp.int32),
            pltpu.VMEM((SEG, 1, CHUNK), jnp.int32),
            pltpu.VMEM((CHUNK, IN_F), jnp.float32),
            pltpu.VMEM((CHUNK, IN_F), jnp.float32),
            pltpu.VMEM((WSEG + 16,), jnp.float32),
            pltpu.VMEM((2 * CHUNK,), jnp.int32),
            pltpu.VMEM((2 * NPT,), jnp.float32),
            pltpu.VMEM_SHARED((NP, IN_F), jnp.float32),
            pltpu.VMEM_SHARED((2 * NP,), jnp.float32),
            pltpu.SemaphoreType.DMA,
            pltpu.SemaphoreType.DMA,
            pltpu.SemaphoreType.DMA,
            pltpu.SemaphoreType.DMA,
        ],
    )


def _ln_body(hp_ref, rs_ref, g_ref, b_ref, o_ref):
    hp0 = hp_ref[0]
    hp1 = hp_ref[1]
    rs = rs_ref[...]

    def _den(r):
        return jnp.where(r == 0.0, 1.0, r)

    h = jnp.concatenate([
        hp0[:, :HID] / _den(rs[0, :, 0:1]),
        hp0[:, HID:] / _den(rs[0, :, 1:2]),
        hp1[:, :HID] / _den(rs[1, :, 0:1]),
        hp1[:, HID:] / _den(rs[1, :, 1:2]),
    ], axis=1)
    mean = jnp.mean(h, axis=1, keepdims=True)
    xc = h - mean
    std = jnp.sqrt(jnp.sum(xc * xc, axis=1, keepdims=True) / (F - 1))
    y = g_ref[...] * xc / (std + EPS) + b_ref[...]
    o_ref[...] = jnp.where(y > 0, y, jnp.exp(jnp.minimum(y, 0.0)) - 1.0)


def _layernorm(hp, rs, gamma, beta):
    return pl.pallas_call(
        _ln_body,
        grid=(N // BN,),
        in_specs=[
            pl.BlockSpec((NC, BN, IN_F), lambda i: (0, i, 0)),
            pl.BlockSpec((NC, BN, 2), lambda i: (0, i, 0)),
            pl.BlockSpec((1, F), lambda i: (0, 0)),
            pl.BlockSpec((1, F), lambda i: (0, 0)),
        ],
        out_specs=pl.BlockSpec((BN, F), lambda i: (i, 0)),
        out_shape=jax.ShapeDtypeStruct((N, F), jnp.float32),
    )(hp, rs, gamma, beta)


def kernel(data, edge, embed, W_w, W_b, a, a2, b2):
    wfull = W_w.transpose(2, 0, 1).reshape(IN_F, F)
    bfull = W_b.reshape(1, F)
    a1 = a[:, 0, :HID]
    a2h = a[:, 0, HID:]
    cols = []
    for c in range(NC):
        for h in (2 * c, 2 * c + 1):
            cols.append(jnp.zeros((F,), jnp.float32).at[h * HID:(h + 1) * HID].set(a1[h]))
        for h in (2 * c, 2 * c + 1):
            cols.append(jnp.zeros((F,), jnp.float32).at[h * HID:(h + 1) * HID].set(a2h[h]))
    a12 = jnp.stack(cols, axis=1)

    e0 = jnp.zeros((EPAD,), jnp.int32).at[:E].set(edge[0])
    e1 = jnp.zeros((EPAD,), jnp.int32).at[:E].set(edge[1])
    e0r = e0.reshape(NS, CPT, 1, CHUNK)
    e1r = e1.reshape(NS, CPT, 1, CHUNK)

    d_out, s_out = _project(embed, wfull, bfull, a12)
    dcat = d_out.reshape(NC * N, IN_F)

    w_edge = _wgt_kernel()(s_out.reshape(NC, 4 * N), e0r, e1r)
    hp, rs = _agg_kernel()(dcat, w_edge, e0r, e1r)
    return _layernorm(hp, rs.reshape(NC, NP, 2),
                      a2.reshape(1, F), b2.reshape(1, F))


# speedup vs baseline: 11.3046x; 1.0004x over previous
import functools
import math

import jax
import jax.numpy as jnp
from jax import lax
from jax.experimental import pallas as pl
from jax.experimental.pallas import tpu as pltpu
from jax.experimental.pallas import tpu_sc as plsc

N = 10000
E = 320000
IN_F = 128
HID = 64
HEADS = 4
ALPHA = 0.2
EPS = 1e-6
SCALE = 1.0 / math.sqrt(HID * HEADS)
F = HEADS * HID

NC = 2
NS = 16
CHUNK = 128
SEG = 20
NSEG = 8
CPT = SEG * NSEG
EPT = CHUNK * CPT
EPAD = EPT * NS
NP = 10240
NPT = NP // NS
WSEG = 2 * SEG * CHUNK

BN = 400


def _proj_body(x_ref, w_ref, b_ref, a12_ref, d_ref, s_ref):
    x = x_ref[...]
    d = jnp.dot(x, w_ref[...], preferred_element_type=jnp.float32) + b_ref[...]
    d_ref[0] = d[:, :IN_F]
    d_ref[1] = d[:, IN_F:]
    s = jnp.dot(d, a12_ref[...], preferred_element_type=jnp.float32)
    s_ref[0] = s[:, :4]
    s_ref[1] = s[:, 4:]


def _project(embed, wfull, bfull, a12):
    return pl.pallas_call(
        _proj_body,
        grid=(N // BN,),
        in_specs=[
            pl.BlockSpec((BN, IN_F), lambda i: (i, 0)),
            pl.BlockSpec((IN_F, F), lambda i: (0, 0)),
            pl.BlockSpec((1, F), lambda i: (0, 0)),
            pl.BlockSpec((F, 8), lambda i: (0, 0)),
        ],
        out_specs=[
            pl.BlockSpec((NC, BN, IN_F), lambda i: (0, i, 0)),
            pl.BlockSpec((NC, BN, 4), lambda i: (0, i, 0)),
        ],
        out_shape=[
            jax.ShapeDtypeStruct((NC, N, IN_F), jnp.float32),
            jax.ShapeDtypeStruct((NC, N, 4), jnp.float32),
        ],
    )(embed, wfull, bfull, a12)


def _wgt_body(s_hbm, e0_hbm, e1_hbm, w_out, s_v, e0s, e1s, wseg):
    c = lax.axis_index("c")
    s = lax.axis_index("s")
    iota16 = lax.iota(jnp.int32, 16)
    pltpu.sync_copy(s_hbm.at[c], s_v)

    def _seg(seg, carry):
        pltpu.sync_copy(e0_hbm.at[s, pl.ds(seg * SEG, SEG)], e0s)
        pltpu.sync_copy(e1_hbm.at[s, pl.ds(seg * SEG, SEG)], e1s)
        ebase = s * EPT + seg * (SEG * CHUNK)

        def _chunk(j, carry2):
            for g in range(CHUNK // 16):
                sl = pl.ds(g * 16, 16)
                e0g4 = e0s[j, 0, sl] * 4
                e1g4 = e1s[j, 0, sl] * 4
                l0 = (plsc.load_gather(s_v, [e0g4])
                      + plsc.load_gather(s_v, [e1g4 + 2]))
                l1 = (plsc.load_gather(s_v, [e0g4 + 1])
                      + plsc.load_gather(s_v, [e1g4 + 3]))
                l0 = jnp.where(l0 > 0, l0, ALPHA * l0) * SCALE
                l1 = jnp.where(l1 > 0, l1, ALPHA * l1) * SCALE
                pos = ebase + j * CHUNK + g * 16 + iota16
                valid = pos < E
                w0 = jnp.where(valid, jnp.exp(l0), 0.0)
                w1 = jnp.where(valid, jnp.exp(l1), 0.0)
                ids2 = (j * CHUNK + g * 16 + iota16) * 2
                plsc.store_scatter(wseg, [ids2], w0)
                plsc.store_scatter(wseg, [ids2 + 1], w1)
            return carry2
        lax.fori_loop(0, SEG, _chunk, 0)
        pltpu.sync_copy(wseg, w_out.at[c, pl.ds(2 * ebase, WSEG)])
        return carry
    lax.fori_loop(0, NSEG, _seg, 0)


@functools.cache
def _wgt_kernel():
    return pl.kernel(
        _wgt_body,
        out_type=jax.ShapeDtypeStruct((NC, 2 * EPAD), jnp.float32),
        mesh=plsc.VectorSubcoreMesh(core_axis_name="c", subcore_axis_name="s",
                                    num_cores=NC, num_subcores=NS),
        compiler_params=pltpu.CompilerParams(needs_layout_passes=False),
        scratch_types=[
            pltpu.VMEM((4 * N,), jnp.float32),
            pltpu.VMEM((SEG, 1, CHUNK), jnp.int32),
            pltpu.VMEM((SEG, 1, CHUNK), jnp.int32),
            pltpu.VMEM((WSEG,), jnp.float32),
        ],
    )


def _agg_body(d_hbm, w_hbm, e0_hbm, e1_hbm, hp_out, rs_out,
              e0s, e1s, rows, rows2, wseg, ibuf, zr1, hp_acc, rs_acc,
              sem, sem2, ssem, ssem2):
    c = lax.axis_index("c")
    s = lax.axis_index("s")
    off = c * N
    row0 = s * NPT
    z16 = jnp.zeros((16,), jnp.float32)
    iota16 = lax.iota(jnp.int32, 16)

    def _zrows(i, carry):
        for k in range(IN_F // 16):
            rows[i, pl.ds(k * 16, 16)] = z16
        return carry
    lax.fori_loop(0, CHUNK, _zrows, 0)

    def _zr1(i, carry):
        zr1[pl.ds(i * 16, 16)] = z16
        return carry
    lax.fori_loop(0, 2 * NPT // 16, _zr1, 0)

    for q in range(NPT // CHUNK):
        pltpu.sync_copy(rows, hp_acc.at[pl.ds(row0 + q * CHUNK, CHUNK)])
    pltpu.sync_copy(zr1, rs_acc.at[pl.ds(2 * row0, 2 * NPT)])
    plsc.subcore_barrier()

    def _seg(seg, carry):
        pltpu.sync_copy(e0_hbm.at[s, pl.ds(seg * SEG, SEG)], e0s)
        pltpu.sync_copy(e1_hbm.at[s, pl.ds(seg * SEG, SEG)], e1s)
        ebase = s * EPT + seg * (SEG * CHUNK)
        pltpu.sync_copy(w_hbm.at[c, pl.ds(2 * ebase, WSEG)],
                        wseg.at[pl.ds(0, WSEG)])

        def _adj(j, carry2):
            for k in range(CHUNK // 16):
                sl = pl.ds(k * 16, 16)
                e1s[j, 0, sl] = e1s[j, 0, sl] + off
            return carry2
        lax.fori_loop(0, SEG, _adj, 0)

        def _compute_scatter(j, buf, ssem):
            for g in range(CHUNK // 16):
                sl = pl.ds(g * 16, 16)
                ids2 = (g * 16 + iota16) * 2
                e0g2 = e0s[j, 0, sl] * 2
                plsc.store_scatter(ibuf, [ids2], e0g2)
                plsc.store_scatter(ibuf, [ids2 + 1], e0g2 + 1)

            def _mul(e, carry3):
                wv = wseg[pl.ds(2 * (j * CHUNK + e), 16)]
                w0s = wv[0]
                w1s = wv[1]
                for k in range(4):
                    sl = pl.ds(k * 16, 16)
                    buf[e, sl] = buf[e, sl] * w0s
                for k in range(4, 8):
                    sl = pl.ds(k * 16, 16)
                    buf[e, sl] = buf[e, sl] * w1s
                return carry3
            lax.fori_loop(0, CHUNK, _mul, 0, unroll=4)

            pltpu.async_copy(buf, hp_acc.at[e0s.at[j, 0]], ssem, add=True)
            pltpu.sync_copy(wseg.at[pl.ds(2 * (j * CHUNK), 2 * CHUNK)],
                            rs_acc.at[ibuf], add=True)

        def _gather(j, buf, gsem):
            pltpu.async_copy(d_hbm.at[e1s.at[j, 0, pl.ds(0, CHUNK // 2)]],
                             buf.at[pl.ds(0, CHUNK // 2)], gsem)
            pltpu.async_copy(d_hbm.at[e1s.at[j, 0, pl.ds(CHUNK // 2, CHUNK // 2)]],
                             buf.at[pl.ds(CHUNK // 2, CHUNK // 2)], gsem)

        def _gwait(buf, gsem):
            pltpu.make_async_copy(d_hbm.at[e1s.at[0, 0]], buf, gsem).wait()

        def _swait(buf, xsem):
            pltpu.make_async_copy(buf, hp_acc.at[e0s.at[0, 0]], xsem).wait()

        _gather(0, rows, sem)
        _gather(1, rows2, sem2)

        def _pair(p, carry2):
            j0 = 2 * p
            j1 = j0 + 1
            _gwait(rows, sem)
            _compute_scatter(j0, rows, ssem)
            _gwait(rows2, sem2)
            _swait(rows, ssem)

            @pl.when(j0 + 2 < SEG)
            def _():
                _gather(j0 + 2, rows, sem)
            _compute_scatter(j1, rows2, ssem2)
            _swait(rows2, ssem2)

            @pl.when(j1 + 2 < SEG)
            def _():
                _gather(j1 + 2, rows2, sem2)
            return carry2
        lax.fori_loop(0, SEG // 2, _pair, 0)
        return carry
    lax.fori_loop(0, NSEG, _seg, 0)

    plsc.subcore_barrier()

    for q in range(NPT // CHUNK):
        r0 = row0 + q * CHUNK
        pltpu.sync_copy(hp_acc.at[pl.ds(r0, CHUNK)], rows)
        pltpu.sync_copy(rows, hp_out.at[c, pl.ds(r0, CHUNK)])
    pltpu.sync_copy(rs_acc.at[pl.ds(2 * row0, 2 * NPT)], zr1)
    pltpu.sync_copy(zr1, rs_out.at[c, pl.ds(2 * row0, 2 * NPT)])


@functools.cache
def _agg_kernel():
    return pl.kernel(
        _agg_body,
        out_type=[
            jax.ShapeDtypeStruct((NC, NP, IN_F), jnp.float32),
            jax.ShapeDtypeStruct((NC, 2 * NP), jnp.float32),
        ],
        mesh=plsc.VectorSubcoreMesh(core_axis_name="c", subcore_axis_name="s",
                                    num_cores=NC, num_subcores=NS),
        compiler_params=pltpu.CompilerParams(needs_layout_passes=False),
        scratch_types=[
            pltpu.VMEM((SEG, 1, CHUNK), jnp.int32),
            pltpu.VMEM((SEG, 1, CHUNK), jnp.int32),
            pltpu.VMEM((CHUNK, IN_F), jnp.float32),
            pltpu.VMEM((CHUNK, IN_F), jnp.float32),
            pltpu.VMEM((WSEG + 16,), jnp.float32),
            pltpu.VMEM((2 * CHUNK,), jnp.int32),
            pltpu.VMEM((2 * NPT,), jnp.float32),
            pltpu.VMEM_SHARED((NP, IN_F), jnp.float32),
            pltpu.VMEM_SHARED((2 * NP,), jnp.float32),
            pltpu.SemaphoreType.DMA,
            pltpu.SemaphoreType.DMA,
            pltpu.SemaphoreType.DMA,
            pltpu.SemaphoreType.DMA,
        ],
    )


def _ln_body(hp_ref, rs_ref, g_ref, b_ref, o_ref):
    hp0 = hp_ref[0]
    hp1 = hp_ref[1]
    rs = rs_ref[...]

    def _den(r):
        return jnp.where(r == 0.0, 1.0, r)

    h = jnp.concatenate([
        hp0[:, :HID] / _den(rs[0, :, 0:1]),
        hp0[:, HID:] / _den(rs[0, :, 1:2]),
        hp1[:, :HID] / _den(rs[1, :, 0:1]),
        hp1[:, HID:] / _den(rs[1, :, 1:2]),
    ], axis=1)
    mean = jnp.mean(h, axis=1, keepdims=True)
    xc = h - mean
    std = jnp.sqrt(jnp.sum(xc * xc, axis=1, keepdims=True) / (F - 1))
    y = g_ref[...] * xc / (std + EPS) + b_ref[...]
    o_ref[...] = jnp.where(y > 0, y, jnp.exp(jnp.minimum(y, 0.0)) - 1.0)


def _layernorm(hp, rs, gamma, beta):
    return pl.pallas_call(
        _ln_body,
        grid=(N // BN,),
        in_specs=[
            pl.BlockSpec((NC, BN, IN_F), lambda i: (0, i, 0)),
            pl.BlockSpec((NC, BN, 2), lambda i: (0, i, 0)),
            pl.BlockSpec((1, F), lambda i: (0, 0)),
            pl.BlockSpec((1, F), lambda i: (0, 0)),
        ],
        out_specs=pl.BlockSpec((BN, F), lambda i: (i, 0)),
        out_shape=jax.ShapeDtypeStruct((N, F), jnp.float32),
    )(hp, rs, gamma, beta)


def kernel(data, edge, embed, W_w, W_b, a, a2, b2):
    wfull = W_w.transpose(2, 0, 1).reshape(IN_F, F)
    bfull = W_b.reshape(1, F)
    a1 = a[:, 0, :HID]
    a2h = a[:, 0, HID:]
    cols = []
    for c in range(NC):
        for h in (2 * c, 2 * c + 1):
            cols.append(jnp.zeros((F,), jnp.float32).at[h * HID:(h + 1) * HID].set(a1[h]))
        for h in (2 * c, 2 * c + 1):
            cols.append(jnp.zeros((F,), jnp.float32).at[h * HID:(h + 1) * HID].set(a2h[h]))
    a12 = jnp.stack(cols, axis=1)

    e0 = jnp.zeros((EPAD,), jnp.int32).at[:E].set(edge[0])
    e1 = jnp.zeros((EPAD,), jnp.int32).at[:E].set(edge[1])
    e0r = e0.reshape(NS, CPT, 1, CHUNK)
    e1r = e1.reshape(NS, CPT, 1, CHUNK)

    d_out, s_out = _project(embed, wfull, bfull, a12)
    dcat = d_out.reshape(NC * N, IN_F)

    w_edge = _wgt_kernel()(s_out.reshape(NC, 4 * N), e0r, e1r)
    hp, rs = _agg_kernel()(dcat, w_edge, e0r, e1r)
    return _layernorm(hp, rs.reshape(NC, NP, 2),
                      a2.reshape(1, F), b2.reshape(1, F))
